# baseline (speedup 1.0000x reference)
import functools

import jax
import jax.numpy as jnp
from jax import lax
from jax.experimental import pallas as pl
from jax.experimental.pallas import tpu as pltpu
from jax.experimental.pallas import tpu_sc as plsc

_VOCAB = 100000
_D = 64
_L = 200
_B = 4096
_NC = 2
_NS = 16
_NW = _NC * _NS
_BL = _B // _NW
_LT = 8
_NLT = _L // _LT
_NBUF = 4


def _positional_encoding():
    even_i = jnp.arange(0, _D, 2).astype(jnp.float32)
    denominator = jnp.power(10000.0, even_i / _D)
    position = jnp.arange(_L, dtype=jnp.float32).reshape(_L, 1)
    even_pe = jnp.sin(position / denominator)
    odd_pe = jnp.cos(position / denominator)
    stacked = jnp.stack([even_pe, odd_pe], axis=2)
    return stacked.reshape(_L, _D)


def _body(tok_hbm, table_hbm, pos_hbm, out_hbm, tok_v, pos_v, rows_v,
          trans_v, gsems, osems):
    wid = lax.axis_index("s") * _NC + lax.axis_index("c")
    pltpu.sync_copy(pos_hbm, pos_v)

    @pl.loop(0, _NLT, step=1)
    def _stage(i):
        pltpu.sync_copy(tok_hbm.at[i, wid], tok_v.at[pl.ds(i * _LT, _LT)])

    def fire_gather(l, b):
        pltpu.async_copy(table_hbm.at[tok_v.at[l]], rows_v[b], gsems[b])

    def drain_gather(l, b):
        pltpu.make_async_copy(
            table_hbm.at[tok_v.at[l]], rows_v[b], gsems[b]
        ).wait()

    def transpose_add(l, b):
        rv, tv = rows_v[b], trans_v[b]
        pos_j = [pos_v[l, pl.ds(j * 16, 16)] for j in range(4)]
        lane = lax.iota(jnp.int32, 16)
        idx_j = [((lane + j * 16) >> 3, (lane + j * 16) & 7)
                 for j in range(4)]

        @plsc.parallel_loop(0, _BL, 1, unroll=4)
        def _row(bl):
            blv = jnp.full((16,), bl, jnp.int32)
            for j in range(4):
                v = rv[bl, pl.ds(j * 16, 16)] + pos_j[j]
                plsc.store_scatter(tv, [idx_j[j][0], idx_j[j][1], blv], v)

    def fire_out(l, b):
        pltpu.async_copy(trans_v[b].at[:, :, pl.ds(0, _BL)],
                         out_hbm.at[l, :, wid], osems[b])

    def drain_out(l, b):
        pltpu.make_async_copy(
            trans_v[b].at[:, :, pl.ds(0, _BL)],
            out_hbm.at[l, :, wid], osems[b]
        ).wait()

    def steady(l, b, fire_next=True, wait_prev=True):
        nb = (b + 1) % _NBUF
        if wait_prev:
            drain_out(l - (_NBUF - 1), nb)
        if fire_next:
            fire_gather(l + 1, nb)
        drain_gather(l, b)
        transpose_add(l, b)
        fire_out(l, b)

    fire_gather(0, 0)
    for l in range(_NBUF - 1):
        steady(l, l % _NBUF, wait_prev=False)
    steady(_NBUF - 1, (_NBUF - 1) % _NBUF)

    @pl.loop(_NBUF, _L - _NBUF, step=_NBUF)
    def _loop(l0):
        for b in range(_NBUF):
            steady(l0 + b, b)

    for l in range(_L - _NBUF, _L - 1):
        steady(l, l % _NBUF)
    steady(_L - 1, (_L - 1) % _NBUF, fire_next=False)
    for l in range(_L - _NBUF + 1, _L):
        drain_out(l, l % _NBUF)


@functools.partial(jax.jit, static_argnames=())
def kernel(tokens, embedding_table):
    tok = tokens.astype(jnp.int32)
    tok4 = tok.T.reshape(_NLT, _LT, _NW, _BL).transpose(0, 2, 1, 3)
    pos = _positional_encoding()
    mesh = plsc.VectorSubcoreMesh(core_axis_name="c", subcore_axis_name="s")
    call = functools.partial(
        pl.kernel,
        out_type=jax.ShapeDtypeStruct((_L, _D // 8, _NW, 8, _BL),
                                      jnp.float32),
        mesh=mesh,
        compiler_params=pltpu.CompilerParams(
            use_tc_tiling_on_sc=False, needs_layout_passes=False),
        scratch_types=[
            pltpu.VMEM((_L, _BL), jnp.int32),
            pltpu.VMEM((_L, _D), jnp.float32),
            [pltpu.VMEM((_BL, _D), jnp.float32) for _ in range(_NBUF)],
            [pltpu.VMEM((_D // 8, 8, _BL + 1), jnp.float32)
             for _ in range(_NBUF)],
            [pltpu.SemaphoreType.DMA for _ in range(_NBUF)],
            [pltpu.SemaphoreType.DMA for _ in range(_NBUF)],
        ],
    )(_body)
    out5 = call(tok4, embedding_table, pos)
    return out5.transpose(2, 4, 0, 1, 3).reshape(_B, _L, _D)


# speedup vs baseline: 1.1441x; 1.1441x over previous
import functools

import jax
import jax.numpy as jnp
from jax import lax
from jax.experimental import pallas as pl
from jax.experimental.pallas import tpu as pltpu
from jax.experimental.pallas import tpu_sc as plsc

_VOCAB = 100000
_D = 64
_L = 200
_B = 4096
_NC = 2
_NS = 16
_NW = _NC * _NS
_BL = _B // _NW
_LT = 8
_NLT = _L // _LT
_NBUF = 4


def _positional_encoding():
    even_i = jnp.arange(0, _D, 2).astype(jnp.float32)
    denominator = jnp.power(10000.0, even_i / _D)
    position = jnp.arange(_L, dtype=jnp.float32).reshape(_L, 1)
    even_pe = jnp.sin(position / denominator)
    odd_pe = jnp.cos(position / denominator)
    stacked = jnp.stack([even_pe, odd_pe], axis=2)
    return stacked.reshape(_L, _D)


def _body(tok_hbm, table_hbm, pos_hbm, out_hbm, tok_v, pos_v, rows_v,
          trans_v, gsems, osems):
    wid = lax.axis_index("s") * _NC + lax.axis_index("c")
    pltpu.sync_copy(pos_hbm, pos_v)

    @pl.loop(0, _NLT, step=1)
    def _stage(i):
        pltpu.sync_copy(tok_hbm.at[i, wid], tok_v.at[pl.ds(i * _LT, _LT)])

    def fire_gather(l, b):
        pltpu.async_copy(table_hbm.at[tok_v.at[l]], rows_v[b], gsems[b])

    def drain_gather(l, b):
        pltpu.make_async_copy(
            table_hbm.at[tok_v.at[l]], rows_v[b], gsems[b]
        ).wait()

    def transpose_add(l, b):
        rv, tv = rows_v[b], trans_v[b]
        pos_j = [pos_v[l, pl.ds(j * 16, 16)] for j in range(4)]
        lane = lax.iota(jnp.int32, 16)
        idx_j = [((lane + j * 16) >> 3, (lane + j * 16) & 7)
                 for j in range(4)]

        @plsc.parallel_loop(0, _BL, 1, unroll=4)
        def _row(bl):
            blv = jnp.full((16,), bl, jnp.int32)
            for j in range(4):
                v = rv[bl, pl.ds(j * 16, 16)] + pos_j[j]
                plsc.store_scatter(tv, [idx_j[j][0], idx_j[j][1], blv], v)

    def fire_out(l, b):
        pltpu.async_copy(trans_v[b].at[:, :, pl.ds(0, _BL)],
                         out_hbm.at[l, :, wid], osems[b])

    def drain_out(l, b):
        pltpu.make_async_copy(
            trans_v[b].at[:, :, pl.ds(0, _BL)],
            out_hbm.at[l, :, wid], osems[b]
        ).wait()

    _AHEAD = 2

    def steady(l, b, fire_next=True, wait_prev=True):
        nb = (b + _AHEAD) % _NBUF
        if wait_prev:
            drain_out(l - (_NBUF - _AHEAD), nb)
        if fire_next:
            fire_gather(l + _AHEAD, nb)
        drain_gather(l, b)
        transpose_add(l, b)
        fire_out(l, b)

    for l in range(_AHEAD):
        fire_gather(l, l)
    for l in range(_NBUF - _AHEAD):
        steady(l, l % _NBUF, wait_prev=False)
    for l in range(_NBUF - _AHEAD, _NBUF):
        steady(l, l % _NBUF)

    @pl.loop(_NBUF, _L - _NBUF, step=_NBUF)
    def _loop(l0):
        for b in range(_NBUF):
            steady(l0 + b, b)

    for l in range(_L - _NBUF, _L - _AHEAD):
        steady(l, l % _NBUF)
    for l in range(_L - _AHEAD, _L):
        steady(l, l % _NBUF, fire_next=False)
    for l in range(_L - (_NBUF - _AHEAD), _L):
        drain_out(l, l % _NBUF)


@functools.partial(jax.jit, static_argnames=())
def kernel(tokens, embedding_table):
    tok = tokens.astype(jnp.int32)
    tok4 = tok.T.reshape(_NLT, _LT, _NW, _BL).transpose(0, 2, 1, 3)
    pos = _positional_encoding()
    mesh = plsc.VectorSubcoreMesh(core_axis_name="c", subcore_axis_name="s")
    call = functools.partial(
        pl.kernel,
        out_type=jax.ShapeDtypeStruct((_L, _D // 8, _NW, 8, _BL),
                                      jnp.float32),
        mesh=mesh,
        compiler_params=pltpu.CompilerParams(
            use_tc_tiling_on_sc=False, needs_layout_passes=False),
        scratch_types=[
            pltpu.VMEM((_L, _BL), jnp.int32),
            pltpu.VMEM((_L, _D), jnp.float32),
            [pltpu.VMEM((_BL, _D), jnp.float32) for _ in range(_NBUF)],
            [pltpu.VMEM((_D // 8, 8, _BL + 1), jnp.float32)
             for _ in range(_NBUF)],
            [pltpu.SemaphoreType.DMA for _ in range(_NBUF)],
            [pltpu.SemaphoreType.DMA for _ in range(_NBUF)],
        ],
    )(_body)
    out5 = call(tok4, embedding_table, pos)
    return out5.transpose(2, 4, 0, 1, 3).reshape(_B, _L, _D)


# speedup vs baseline: 1.1453x; 1.0011x over previous
import functools

import jax
import jax.numpy as jnp
from jax import lax
from jax.experimental import pallas as pl
from jax.experimental.pallas import tpu as pltpu
from jax.experimental.pallas import tpu_sc as plsc

_VOCAB = 100000
_D = 64
_L = 200
_B = 4096
_NC = 2
_NS = 16
_NW = _NC * _NS
_BL = _B // _NW
_LT = 8
_NLT = _L // _LT
_NBUF = 5


def _positional_encoding():
    even_i = jnp.arange(0, _D, 2).astype(jnp.float32)
    denominator = jnp.power(10000.0, even_i / _D)
    position = jnp.arange(_L, dtype=jnp.float32).reshape(_L, 1)
    even_pe = jnp.sin(position / denominator)
    odd_pe = jnp.cos(position / denominator)
    stacked = jnp.stack([even_pe, odd_pe], axis=2)
    return stacked.reshape(_L, _D)


def _body(tok_hbm, table_hbm, pos_hbm, out_hbm, tok_v, pos_v, rows_v,
          trans_v, gsems, osems):
    wid = lax.axis_index("s") * _NC + lax.axis_index("c")
    pltpu.sync_copy(pos_hbm, pos_v)
    lane = lax.iota(jnp.int32, 16)
    idx_j = [((lane + j * 16) >> 3, (lane + j * 16) & 7) for j in range(4)]

    @pl.loop(0, _NLT, step=1)
    def _stage(i):
        pltpu.sync_copy(tok_hbm.at[i, wid], tok_v.at[pl.ds(i * _LT, _LT)])

    def fire_gather(l, b):
        pltpu.async_copy(table_hbm.at[tok_v.at[l]], rows_v[b], gsems[b])

    def drain_gather(l, b):
        pltpu.make_async_copy(
            table_hbm.at[tok_v.at[l]], rows_v[b], gsems[b]
        ).wait()

    def transpose_add(l, b):
        rv, tv = rows_v[b], trans_v[b]
        pos_j = [pos_v[l, pl.ds(j * 16, 16)] for j in range(4)]

        @plsc.parallel_loop(0, _BL, 1, unroll=4)
        def _row(bl):
            blv = jnp.full((16,), bl, jnp.int32)
            for j in range(4):
                v = rv[bl, pl.ds(j * 16, 16)] + pos_j[j]
                plsc.store_scatter(tv, [idx_j[j][0], idx_j[j][1], blv], v)

    def fire_out(l, b):
        pltpu.async_copy(trans_v[b].at[:, :, pl.ds(0, _BL)],
                         out_hbm.at[l, :, wid], osems[b])

    def drain_out(l, b):
        pltpu.make_async_copy(
            trans_v[b].at[:, :, pl.ds(0, _BL)],
            out_hbm.at[l, :, wid], osems[b]
        ).wait()

    _AHEAD = 2

    def steady(l, b, fire_next=True, wait_prev=True):
        nb = (b + _AHEAD) % _NBUF
        if wait_prev:
            drain_out(l - (_NBUF - _AHEAD), nb)
        if fire_next:
            fire_gather(l + _AHEAD, nb)
        drain_gather(l, b)
        transpose_add(l, b)
        fire_out(l, b)

    for l in range(_AHEAD):
        fire_gather(l, l)
    for l in range(_NBUF - _AHEAD):
        steady(l, l % _NBUF, wait_prev=False)
    for l in range(_NBUF - _AHEAD, _NBUF):
        steady(l, l % _NBUF)

    @pl.loop(_NBUF, _L - _NBUF, step=_NBUF)
    def _loop(l0):
        for b in range(_NBUF):
            steady(l0 + b, b)

    for l in range(_L - _NBUF, _L - _AHEAD):
        steady(l, l % _NBUF)
    for l in range(_L - _AHEAD, _L):
        steady(l, l % _NBUF, fire_next=False)
    for l in range(_L - (_NBUF - _AHEAD), _L):
        drain_out(l, l % _NBUF)


@functools.partial(jax.jit, static_argnames=())
def kernel(tokens, embedding_table):
    tok = tokens.astype(jnp.int32)
    tok4 = tok.T.reshape(_NLT, _LT, _NW, _BL).transpose(0, 2, 1, 3)
    pos = _positional_encoding()
    mesh = plsc.VectorSubcoreMesh(core_axis_name="c", subcore_axis_name="s")
    call = functools.partial(
        pl.kernel,
        out_type=jax.ShapeDtypeStruct((_L, _D // 8, _NW, 8, _BL),
                                      jnp.float32),
        mesh=mesh,
        compiler_params=pltpu.CompilerParams(
            use_tc_tiling_on_sc=False, needs_layout_passes=False),
        scratch_types=[
            pltpu.VMEM((_L, _BL), jnp.int32),
            pltpu.VMEM((_L, _D), jnp.float32),
            [pltpu.VMEM((_BL, _D), jnp.float32) for _ in range(_NBUF)],
            [pltpu.VMEM((_D // 8, 8, _BL + 1), jnp.float32)
             for _ in range(_NBUF)],
            [pltpu.SemaphoreType.DMA for _ in range(_NBUF)],
            [pltpu.SemaphoreType.DMA for _ in range(_NBUF)],
        ],
    )(_body)
    out5 = call(tok4, embedding_table, pos)
    return out5.transpose(2, 4, 0, 1, 3).reshape(_B, _L, _D)


# speedup vs baseline: 1.1986x; 1.0465x over previous
import functools

import jax
import jax.numpy as jnp
from jax import lax
from jax.experimental import pallas as pl
from jax.experimental.pallas import tpu as pltpu
from jax.experimental.pallas import tpu_sc as plsc

_VOCAB = 100000
_D = 64
_L = 200
_B = 4096
_NC = 2
_NS = 16
_NW = _NC * _NS
_BL = _B // _NW
_LT = 8
_NLT = _L // _LT
_NBUF = 5


def _positional_encoding():
    even_i = jnp.arange(0, _D, 2).astype(jnp.float32)
    denominator = jnp.power(10000.0, even_i / _D)
    position = jnp.arange(_L, dtype=jnp.float32).reshape(_L, 1)
    even_pe = jnp.sin(position / denominator)
    odd_pe = jnp.cos(position / denominator)
    stacked = jnp.stack([even_pe, odd_pe], axis=2)
    return stacked.reshape(_L, _D)


def _body(tok_hbm, table_hbm, pos_hbm, out_hbm, tok_v, pos_v, rows_v,
          trans_v, gsems, osems):
    wid = lax.axis_index("s") * _NC + lax.axis_index("c")
    pltpu.sync_copy(pos_hbm, pos_v)
    lane = lax.iota(jnp.int32, 16)
    idx_j = [((lane + j * 16) >> 3, (lane + j * 16) & 7) for j in range(4)]

    pltpu.sync_copy(tok_hbm.at[:, wid], tok_v)

    def fire_gather(l, b):
        pltpu.async_copy(table_hbm.at[tok_v.at[l // _LT, l % _LT]],
                         rows_v[b], gsems[b])

    def drain_gather(l, b):
        pltpu.make_async_copy(
            table_hbm.at[tok_v.at[l // _LT, l % _LT]], rows_v[b], gsems[b]
        ).wait()

    def transpose_add(l, b):
        rv, tv = rows_v[b], trans_v[b]
        pos_j = [pos_v[l, pl.ds(j * 16, 16)] for j in range(4)]

        @plsc.parallel_loop(0, _BL, 1, unroll=4)
        def _row(bl):
            blv = jnp.full((16,), bl, jnp.int32)
            for j in range(4):
                v = rv[bl, pl.ds(j * 16, 16)] + pos_j[j]
                plsc.store_scatter(tv, [idx_j[j][0], idx_j[j][1], blv], v)

    def fire_out(l, b):
        pltpu.async_copy(trans_v[b].at[:, :, pl.ds(0, _BL)],
                         out_hbm.at[l, :, wid], osems[b])

    def drain_out(l, b):
        pltpu.make_async_copy(
            trans_v[b].at[:, :, pl.ds(0, _BL)],
            out_hbm.at[l, :, wid], osems[b]
        ).wait()

    _AHEAD = 2

    def steady(l, b, fire_next=True, wait_prev=True):
        nb = (b + _AHEAD) % _NBUF
        if wait_prev:
            drain_out(l - (_NBUF - _AHEAD), nb)
        if fire_next:
            fire_gather(l + _AHEAD, nb)
        drain_gather(l, b)
        transpose_add(l, b)
        fire_out(l, b)

    for l in range(_AHEAD):
        fire_gather(l, l)
    for l in range(_NBUF - _AHEAD):
        steady(l, l % _NBUF, wait_prev=False)
    for l in range(_NBUF - _AHEAD, _NBUF):
        steady(l, l % _NBUF)

    @pl.loop(_NBUF, _L - _NBUF, step=_NBUF)
    def _loop(l0):
        for b in range(_NBUF):
            steady(l0 + b, b)

    for l in range(_L - _NBUF, _L - _AHEAD):
        steady(l, l % _NBUF)
    for l in range(_L - _AHEAD, _L):
        steady(l, l % _NBUF, fire_next=False)
    for l in range(_L - (_NBUF - _AHEAD), _L):
        drain_out(l, l % _NBUF)


@functools.partial(jax.jit, static_argnames=())
def kernel(tokens, embedding_table):
    tok = tokens.astype(jnp.int32)
    tok4 = tok.T.reshape(_NLT, _LT, _NW, _BL).transpose(0, 2, 1, 3)
    pos = _positional_encoding()
    mesh = plsc.VectorSubcoreMesh(core_axis_name="c", subcore_axis_name="s")
    call = functools.partial(
        pl.kernel,
        out_type=jax.ShapeDtypeStruct((_L, _D // 8, _NW, 8, _BL),
                                      jnp.float32),
        mesh=mesh,
        compiler_params=pltpu.CompilerParams(
            use_tc_tiling_on_sc=False, needs_layout_passes=False),
        scratch_types=[
            pltpu.VMEM((_NLT, _LT, _BL), jnp.int32),
            pltpu.VMEM((_L, _D), jnp.float32),
            [pltpu.VMEM((_BL, _D), jnp.float32) for _ in range(_NBUF)],
            [pltpu.VMEM((_D // 8, 8, _BL + 1), jnp.float32)
             for _ in range(_NBUF)],
            [pltpu.SemaphoreType.DMA for _ in range(_NBUF)],
            [pltpu.SemaphoreType.DMA for _ in range(_NBUF)],
        ],
    )(_body)
    out5 = call(tok4, embedding_table, pos)
    return out5.transpose(2, 4, 0, 1, 3).reshape(_B, _L, _D)
